# baseline (speedup 1.0000x reference)
import functools

import jax
import jax.numpy as jnp
from jax import lax
from jax.experimental import pallas as pl
from jax.experimental.pallas import tpu as pltpu
from jax.experimental.pallas import tpu_sc as plsc

_NUM_BS = 256
_NUM_UE = 8
_EMB_DIM = 64
_BATCH = 16384
_PAIRS = _NUM_BS * _NUM_UE

_INFO = plsc.get_sparse_core_info()
_NC = _INFO.num_cores
_NS = _INFO.num_subcores
_L = _INFO.num_lanes
_NW = _NC * _NS
_BPW = _BATCH // _NW
_CHUNK = 128
_NCHUNK = _BPW // _CHUNK

_mesh = plsc.VectorSubcoreMesh(core_axis_name="c", subcore_axis_name="s")


@functools.partial(
    pl.kernel,
    out_type=jax.ShapeDtypeStruct((_EMB_DIM, _BATCH), jnp.float32),
    mesh=_mesh,
    scratch_types=[
        pltpu.VMEM((_BPW,), jnp.int32),
        pltpu.VMEM((_BPW,), jnp.int32),
        pltpu.VMEM((_NCHUNK, _CHUNK), jnp.int32),
        pltpu.VMEM((_BPW, _EMB_DIM), jnp.float32),
        pltpu.VMEM((_EMB_DIM, _BPW), jnp.float32),
        pltpu.SemaphoreType.DMA,
        pltpu.SemaphoreType.DMA((_NCHUNK,)),
        pltpu.SemaphoreType.DMA,
    ],
    compiler_params=pltpu.CompilerParams(use_tc_tiling_on_sc=False,
                                         needs_layout_passes=False),
)
def _gather_kernel(bs_hbm, ue_hbm, tab_hbm, out_hbm,
                   bs_v, ue_v, idx_v, rows_v, outt_v,
                   sem_in, sem_g, sem_o):
    wid = lax.axis_index("s") * _NC + lax.axis_index("c")
    base = wid * _BPW
    iota = lax.iota(jnp.int32, _L)
    cp_b = pltpu.async_copy(bs_hbm.at[pl.ds(base, _BPW)], bs_v, sem_in)
    cp_u = pltpu.async_copy(ue_hbm.at[pl.ds(base, _BPW)], ue_v, sem_in)
    cp_b.wait()
    cp_u.wait()
    gathers = []
    for j in range(_NCHUNK):
        for c in range(_CHUNK // _L):
            i = j * (_CHUNK // _L) + c
            b = bs_v[pl.ds(i * _L, _L)]
            u = ue_v[pl.ds(i * _L, _L)]
            idx_v[j, pl.ds(c * _L, _L)] = b * _NUM_UE + u
        gathers.append(
            pltpu.async_copy(tab_hbm.at[idx_v.at[j]],
                             rows_v.at[pl.ds(j * _CHUNK, _CHUNK)],
                             sem_g.at[j]))
    outs = []
    for j in range(_NCHUNK):
        gathers[j].wait()
        for blk in range(_CHUNK // _L):
            i0 = j * _CHUNK + blk * _L
            srow = i0 + iota

            @plsc.parallel_loop(0, _EMB_DIM, 1, unroll=8)
            def _transpose(c, i0=i0, srow=srow):
                col = jnp.full((_L,), c, jnp.int32)
                outt_v[c, pl.ds(i0, _L)] = plsc.load_gather(rows_v,
                                                            [srow, col])
        outs.append(
            pltpu.async_copy(
                outt_v.at[:, pl.ds(j * _CHUNK, _CHUNK)],
                out_hbm.at[:, pl.ds(base + j * _CHUNK, _CHUNK)],
                sem_o))
    for cp in outs:
        cp.wait()


def kernel(bs_antenna_indices, ue_antenna_indices, embeddings):
    depad = embeddings.reshape(_PAIRS // 2, 2 * _EMB_DIM)
    depad = lax.optimization_barrier(depad)
    flat_table = depad.reshape(_PAIRS, _EMB_DIM)
    out_t = _gather_kernel(bs_antenna_indices.astype(jnp.int32),
                           ue_antenna_indices.astype(jnp.int32),
                           flat_table)
    return out_t.T


# speedup vs baseline: 1.5105x; 1.5105x over previous
import functools

import jax
import jax.numpy as jnp
from jax import lax
from jax.experimental import pallas as pl
from jax.experimental.pallas import tpu as pltpu
from jax.experimental.pallas import tpu_sc as plsc

_NUM_BS = 256
_NUM_UE = 8
_EMB_DIM = 64
_BATCH = 16384
_PAIRS = _NUM_BS * _NUM_UE

_INFO = plsc.get_sparse_core_info()
_NC = _INFO.num_cores
_NS = _INFO.num_subcores
_L = _INFO.num_lanes
_NW = _NC * _NS
_BPW = _BATCH // _NW
_CHUNK = 128
_NCHUNK = _BPW // _CHUNK

_mesh = plsc.VectorSubcoreMesh(core_axis_name="c", subcore_axis_name="s")


@functools.partial(
    pl.kernel,
    out_type=jax.ShapeDtypeStruct((_EMB_DIM, _BATCH), jnp.float32),
    mesh=_mesh,
    scratch_types=[
        pltpu.VMEM((_BPW,), jnp.int32),
        pltpu.VMEM((_BPW,), jnp.int32),
        pltpu.VMEM((_NCHUNK, _CHUNK), jnp.int32),
        pltpu.VMEM((_BPW, _EMB_DIM), jnp.float32),
        pltpu.VMEM((_EMB_DIM, _BPW), jnp.float32),
        pltpu.SemaphoreType.DMA,
        pltpu.SemaphoreType.DMA((_NCHUNK,)),
        pltpu.SemaphoreType.DMA,
    ],
    compiler_params=pltpu.CompilerParams(use_tc_tiling_on_sc=False,
                                         needs_layout_passes=False),
)
def _gather_kernel(bs_hbm, ue_hbm, tab_hbm, out_hbm,
                   bs_v, ue_v, idx_v, rows_v, outt_v,
                   sem_in, sem_g, sem_o):
    wid = lax.axis_index("s") * _NC + lax.axis_index("c")
    base = wid * _BPW
    iota = lax.iota(jnp.int32, _L)
    cp_b = pltpu.async_copy(bs_hbm.at[pl.ds(base, _BPW)], bs_v, sem_in)
    cp_u = pltpu.async_copy(ue_hbm.at[pl.ds(base, _BPW)], ue_v, sem_in)
    cp_b.wait()
    cp_u.wait()
    gathers = []
    for j in range(_NCHUNK):
        for c in range(_CHUNK // _L):
            i = j * (_CHUNK // _L) + c
            b = bs_v[pl.ds(i * _L, _L)]
            u = ue_v[pl.ds(i * _L, _L)]
            idx_v[j, pl.ds(c * _L, _L)] = b * _NUM_UE + u
        gathers.append(
            pltpu.async_copy(tab_hbm.at[idx_v.at[j]],
                             rows_v.at[pl.ds(j * _CHUNK, _CHUNK)],
                             sem_g.at[j]))
    outs = []
    for j in range(_NCHUNK):
        gathers[j].wait()

        @plsc.parallel_loop(0, _CHUNK // _L, 1)
        def _blocks(blk, j=j):
            i0 = j * _CHUNK + blk * _L
            srow = i0 + iota
            colv = i0 + iota

            @plsc.parallel_loop(0, _L, 1, unroll=2)
            def _diag(d, srow=srow, colv=colv):
                diag = (iota + d) & (_L - 1)
                for c0 in range(0, _EMB_DIM, _L):
                    rc = c0 + diag
                    vals = plsc.load_gather(rows_v, [srow, rc])
                    plsc.store_scatter(outt_v, [rc, colv], vals)
        outs.append(
            pltpu.async_copy(
                outt_v.at[:, pl.ds(j * _CHUNK, _CHUNK)],
                out_hbm.at[:, pl.ds(base + j * _CHUNK, _CHUNK)],
                sem_o))
    for cp in outs:
        cp.wait()


def kernel(bs_antenna_indices, ue_antenna_indices, embeddings):
    depad = embeddings.reshape(_PAIRS // 2, 2 * _EMB_DIM)
    depad = lax.optimization_barrier(depad)
    flat_table = depad.reshape(_PAIRS, _EMB_DIM)
    out_t = _gather_kernel(bs_antenna_indices.astype(jnp.int32),
                           ue_antenna_indices.astype(jnp.int32),
                           flat_table)
    return out_t.T
